# Initial kernel scaffold; baseline (speedup 1.0000x reference)
import jax
import jax.numpy as jnp
from jax.experimental import pallas as pl


def kernel(node_features, rigids_rot, rigids_trans, edge_features, edge_index, new_seq_edge_inputs, seq_edge_features, seq_edge_index, res_mask, noising_mask, params):
    raise NotImplementedError("write your pallas kernel here")


# speedup vs baseline: 1.8796x; 1.8796x over previous
import functools
import math

import jax
import jax.numpy as jnp
import numpy as np
from jax.experimental import pallas as pl
from jax.experimental.pallas import tpu as pltpu

N = 10000
E = 160000
SE = 20000
C_S = 128
C_Z = 128
C_HID = 16
H = 4
P_QK = 4
P_V = 8

_EPS_LN = 1e-5


def _linear(p, x):
    return x @ p['w'] + p['b']


def _layernorm(p, x, eps=_EPS_LN):
    m = jnp.mean(x, -1, keepdims=True)
    v = jnp.var(x, -1, keepdims=True)
    return (x - m) / jnp.sqrt(v + eps) * p['g'] + p['b']


def _quat_to_rot(q):
    w, x, y, z = q[..., 0], q[..., 1], q[..., 2], q[..., 3]
    r00 = 1 - 2 * (y * y + z * z); r01 = 2 * (x * y - z * w); r02 = 2 * (x * z + y * w)
    r10 = 2 * (x * y + z * w); r11 = 1 - 2 * (x * x + z * z); r12 = 2 * (y * z - x * w)
    r20 = 2 * (x * z - y * w); r21 = 2 * (y * z + x * w); r22 = 1 - 2 * (x * x + y * y)
    return jnp.stack([jnp.stack([r00, r01, r02], -1), jnp.stack([r10, r11, r12], -1), jnp.stack([r20, r21, r22], -1)], -2)


def _mlp3_ln_body(has_res, x1_ref, gs_ref, gd_ref, res_ref,
                  wa_ref, b1_ref, w2_ref, b2_ref, w3_ref, b3_ref,
                  g_ref, be_ref, o_ref):
    h = jnp.dot(x1_ref[...], wa_ref[...], preferred_element_type=jnp.float32)
    h = h + gs_ref[...] + gd_ref[...] + b1_ref[...]
    h = jnp.maximum(h, 0.0)
    h = jnp.dot(h, w2_ref[...], preferred_element_type=jnp.float32) + b2_ref[...]
    h = jnp.maximum(h, 0.0)
    h = jnp.dot(h, w3_ref[...], preferred_element_type=jnp.float32) + b3_ref[...]
    if has_res:
        h = h + res_ref[...]
    m = jnp.mean(h, -1, keepdims=True)
    hc = h - m
    v = jnp.mean(hc * hc, -1, keepdims=True)
    o_ref[...] = hc * jax.lax.rsqrt(v + _EPS_LN) * g_ref[...] + be_ref[...]


def _edge_mlp(ef, gsrc, gdst, layers, ln, residual=None, block=2000):
    e = ef.shape[0]
    assert e % block == 0
    wa = layers[0]['w'][:C_S]
    wb = layers[0]['w'][C_S:C_Z + 4]
    b1 = layers[0]['b'][None, :]
    x1 = ef[:, :C_S]
    gsrc = gsrc + ef[:, C_S:] @ wb
    grid = e // block
    row = lambda i: (i, 0)
    fixed = lambda i: (0, 0)
    has_res = residual is not None
    res = residual if has_res else jnp.zeros((1, C_Z), jnp.float32)
    res_spec = (pl.BlockSpec((block, C_Z), row) if has_res
                else pl.BlockSpec((1, C_Z), fixed))
    out = pl.pallas_call(
        functools.partial(_mlp3_ln_body, has_res),
        grid=(grid,),
        in_specs=[
            pl.BlockSpec((block, C_S), row),
            pl.BlockSpec((block, C_Z), row),
            pl.BlockSpec((block, C_Z), row),
            res_spec,
            pl.BlockSpec((C_S, C_Z), fixed),
            pl.BlockSpec((1, C_Z), fixed),
            pl.BlockSpec((C_Z, C_Z), fixed),
            pl.BlockSpec((1, C_Z), fixed),
            pl.BlockSpec((C_Z, C_Z), fixed),
            pl.BlockSpec((1, C_Z), fixed),
            pl.BlockSpec((1, C_Z), fixed),
            pl.BlockSpec((1, C_Z), fixed),
        ],
        out_specs=pl.BlockSpec((block, C_Z), row),
        out_shape=jax.ShapeDtypeStruct((e, C_Z), jnp.float32),
    )(x1, gsrc, gdst, res,
      wa, b1, layers[1]['w'], layers[1]['b'][None, :],
      layers[2]['w'], layers[2]['b'][None, :],
      ln['g'][None, :], ln['b'][None, :])
    return out


def _ipa(p, s, z, edge_index, rot, trans, mask):
    src = edge_index[0]; dst = edge_index[1]
    n = s.shape[0]
    q_flat = _linear(p['q'], s)
    kv_flat = _linear(p['kv'], s)
    q_pts = _linear(p['q_pts'], s).reshape(n, H * P_QK, 3)
    q_pts_g = (jnp.einsum('nij,npj->npi', rot, q_pts) + trans[:, None, :])
    kv_pts = _linear(p['kv_pts'], s).reshape(n, H * (P_QK + P_V), 3)
    kv_pts_g = (jnp.einsum('nij,npj->npi', rot, kv_pts) + trans[:, None, :])
    kv_pts_flat = kv_pts_g.reshape(n, H * (P_QK + P_V) * 3)
    k_pts_flat = kv_pts_flat.reshape(n, H, P_QK + P_V, 3)[:, :, :P_QK].reshape(n, H * P_QK * 3)
    v_pts_flat = kv_pts_flat.reshape(n, H, P_QK + P_V, 3)[:, :, P_QK:].reshape(n, H * P_V * 3)
    q_pts_flat = q_pts_g.reshape(n, H * P_QK * 3)

    b = _linear(p['bias'], z)

    qd = q_flat[dst].reshape(-1, H, C_HID)
    kvm = jnp.concatenate([kv_flat, mask[:, None]], axis=1)
    kvs = kvm[src]
    kv_e = kvs[:, :2 * H * C_HID].reshape(-1, H, 2 * C_HID)
    k_e = kv_e[:, :, :C_HID]
    v_e = kv_e[:, :, C_HID:]
    m_e = kvs[:, -1]
    qp_e = q_pts_flat[dst].reshape(-1, H, P_QK, 3)
    kp_e = k_pts_flat[src].reshape(-1, H, P_QK, 3)
    vp_e = v_pts_flat[src].reshape(-1, H, P_V, 3)

    a = jnp.einsum('ehc,ehc->eh', qd, k_e) * math.sqrt(1.0 / (3 * C_HID))
    a = a + b * math.sqrt(1.0 / 3)
    d2 = jnp.sum((qp_e - kp_e) ** 2, axis=(-1, -2))
    head_w = jax.nn.softplus(p['head_w']) * math.sqrt(1.0 / (3 * (P_QK * 9.0 / 2)))
    a = a - 0.5 * head_w[None, :] * d2
    a = a + (m_e - 1.0)[:, None] * 1e9
    amax = jax.ops.segment_max(a, dst, num_segments=n)
    amax = jnp.where(jnp.isfinite(amax), amax, 0.0)
    ea = jnp.exp(a - amax[dst])
    denom = jax.ops.segment_sum(ea, dst, num_segments=n) + 1e-9
    w = ea / denom[dst]
    o = jax.ops.segment_sum(w[..., None] * v_e, dst, num_segments=n)
    o_pt = jax.ops.segment_sum((w[..., None, None] * vp_e).reshape(-1, H * P_V * 3), dst, num_segments=n)
    o_pt = o_pt.reshape(n, H, P_V, 3)
    o_pt = jnp.einsum('nji,nhpj->nhpi', rot, o_pt - trans[:, None, None, :])
    o_pt_norm = jnp.sqrt(jnp.sum(o_pt ** 2, axis=-1) + 1e-8)
    o_pair = jax.ops.segment_sum(w[..., None] * z[:, None, :], dst, num_segments=n)
    feats = jnp.concatenate([o.reshape(n, -1), o_pt.reshape(n, -1), o_pt_norm.reshape(n, -1), o_pair.reshape(n, -1)], axis=-1)
    return _linear(p['out'], feats)


def _mlp3(layers, x):
    x = jax.nn.relu(_linear(layers[0], x))
    x = jax.nn.relu(_linear(layers[1], x))
    return _linear(layers[2], x)


def _compose_q_update(rot, trans, upd):
    bcd = upd[:, :3]; t = upd[:, 3:]
    q = jnp.concatenate([jnp.ones((upd.shape[0], 1), upd.dtype), bcd], axis=-1)
    q = q / jnp.linalg.norm(q, axis=-1, keepdims=True)
    r_upd = _quat_to_rot(q)
    new_rot = jnp.einsum('nij,njk->nik', rot, r_upd)
    new_trans = jnp.einsum('nij,nj->ni', rot, t) + trans
    return new_rot, new_trans


def kernel(node_features, rigids_rot, rigids_trans, edge_features, edge_index,
           new_seq_edge_inputs, seq_edge_features, seq_edge_index, res_mask,
           noising_mask, params):
    src = edge_index[0]; dst = edge_index[1]
    s_src = seq_edge_index[0]; s_dst = seq_edge_index[1]
    keep = (res_mask > 0).astype(jnp.float32)
    nmask = (noising_mask > 0).astype(jnp.float32)

    w1e = params['edge_embed'][0]['w']
    ps_e = node_features @ w1e[C_Z + 4:C_Z + 4 + C_S]
    pd_e = node_features @ w1e[C_Z + 4 + C_S:]
    edge_out = _edge_mlp(edge_features, ps_e[src], pd_e[dst],
                         params['edge_embed'], params['edge_embed_ln'])

    w1s = params['seq_edge_update'][0]['w']
    ps_s = node_features @ w1s[C_Z + 4:C_Z + 4 + C_S]
    pd_s = node_features @ w1s[C_Z + 4 + C_S:]
    seq_edge_feat = _edge_mlp(new_seq_edge_inputs, ps_s[s_src], pd_s[s_dst],
                              params['seq_edge_update'], params['seq_edge_ln'],
                              residual=seq_edge_features, block=2000)

    upd = _ipa(params['ipa_spatial'], node_features, edge_out, edge_index,
               rigids_rot, rigids_trans, keep)
    node = _layernorm(params['ln_s1'], node_features + upd * keep[:, None])
    upd = _ipa(params['ipa_seq'], node, seq_edge_feat, seq_edge_index,
               rigids_rot, rigids_trans, keep)
    node = _layernorm(params['ln_s2'], node + upd * keep[:, None])

    h = jax.nn.relu(_linear(params['trans1'], node))
    h = jax.nn.relu(_linear(params['trans2'], h))
    h = _linear(params['trans3'], h)
    node = _layernorm(params['trans_ln'], node + h)
    node = node * keep[:, None]

    rig_upd = _linear(params['bb_update'], node * nmask[:, None]) * nmask[:, None]
    new_rot, new_trans = _compose_q_update(rigids_rot, rigids_trans, rig_upd)

    ne = _linear(params['et_init'], node)
    x = jnp.concatenate([seq_edge_feat, ne[s_src], ne[s_dst]], axis=-1)
    h2 = jax.nn.relu(_linear(params['et_trunk1'], x))
    h2 = jax.nn.relu(_linear(params['et_trunk2'], h2))
    seq_edge_out = _layernorm(params['et_ln'], _linear(params['et_final'], h2 + x))
    return node, new_rot, new_trans, edge_out, seq_edge_out
